# Initial kernel scaffold; baseline (speedup 1.0000x reference)
import jax
import jax.numpy as jnp
from jax.experimental import pallas as pl


def kernel(nodes, emb_weight):
    raise NotImplementedError("write your pallas kernel here")


# speedup vs baseline: 1.8460x; 1.8460x over previous
import functools

import jax
import jax.numpy as jnp
from jax import lax
from jax.experimental import pallas as pl
from jax.experimental.pallas import tpu as pltpu
from jax.experimental.pallas import tpu_sc as plsc

NUM_NODES = 1000000
EMBED_DIM = 64
BATCH = 16384
HIST = 50

_B = BATCH * HIST
_LANE = 128
_ROWS_PER_CHUNK = 1024
_GPC = _ROWS_PER_CHUNK // _LANE


def _make_gather(nw: int):
    b_per_w = _B // nw
    n_chunks = b_per_w // _ROWS_PER_CHUNK
    mesh = plsc.VectorSubcoreMesh(core_axis_name="c", subcore_axis_name="s")

    @functools.partial(
        pl.kernel,
        out_type=jax.ShapeDtypeStruct((_B, EMBED_DIM), jnp.float32),
        mesh=mesh,
        scratch_types=[
            pltpu.VMEM((_GPC, _LANE), jnp.int32),
            pltpu.VMEM((_ROWS_PER_CHUNK, EMBED_DIM), jnp.float32),
            pltpu.SemaphoreType.DMA,
        ],
        compiler_params=pltpu.CompilerParams(use_tc_tiling_on_sc=False),
    )
    def gather_kernel(idx_hbm, table_hbm, out_hbm, idx_v, rows_v, sem):
        nc = lax.axis_size("c")
        wid = lax.axis_index("s") * nc + lax.axis_index("c")

        def body(i, carry):
            base = pl.multiple_of(wid * b_per_w + i * _ROWS_PER_CHUNK, 1024)
            idx_row0 = pl.multiple_of(
                wid * (b_per_w // _LANE) + i * _GPC, 8
            )
            pltpu.sync_copy(idx_hbm.at[pl.ds(idx_row0, _GPC), :], idx_v)
            copies = [
                pltpu.async_copy(
                    table_hbm.at[idx_v.at[j]],
                    rows_v.at[pl.ds(j * _LANE, _LANE), :],
                    sem,
                )
                for j in range(_GPC)
            ]
            for c in copies:
                c.wait()
            pltpu.sync_copy(rows_v, out_hbm.at[pl.ds(base, _ROWS_PER_CHUNK), :])
            return carry

        lax.fori_loop(0, n_chunks, body, 0)

    return gather_kernel


def kernel(nodes, emb_weight):
    info = plsc.get_sparse_core_info()
    nw = info.num_cores * info.num_subcores
    idx2d = nodes.reshape(_B // _LANE, _LANE)
    out = _make_gather(nw)(idx2d, emb_weight)
    return out.reshape(BATCH, HIST, EMBED_DIM)


# speedup vs baseline: 1.8705x; 1.0133x over previous
import functools

import jax
import jax.numpy as jnp
from jax import lax
from jax.experimental import pallas as pl
from jax.experimental.pallas import tpu as pltpu
from jax.experimental.pallas import tpu_sc as plsc

NUM_NODES = 1000000
EMBED_DIM = 64
BATCH = 16384
HIST = 50

_B = BATCH * HIST
_LANE = 128
_CH = 512
_GPC = _CH // _LANE
_IPP = 2 * _GPC


def _make_gather(nw: int):
    b_per_w = _B // nw
    n_chunks = b_per_w // _CH
    n_pairs = n_chunks // 2
    idx_rows_per_w = b_per_w // _LANE
    mesh = plsc.VectorSubcoreMesh(core_axis_name="c", subcore_axis_name="s")

    @functools.partial(
        pl.kernel,
        out_type=jax.ShapeDtypeStruct((_B, EMBED_DIM), jnp.float32),
        mesh=mesh,
        scratch_types=[
            pltpu.VMEM((_IPP, _LANE), jnp.int32),
            pltpu.VMEM((_IPP, _LANE), jnp.int32),
            pltpu.VMEM((_CH, EMBED_DIM), jnp.float32),
            pltpu.VMEM((_CH, EMBED_DIM), jnp.float32),
            pltpu.SemaphoreType.DMA,
            pltpu.SemaphoreType.DMA,
            pltpu.SemaphoreType.DMA,
            pltpu.SemaphoreType.DMA,
            pltpu.SemaphoreType.DMA,
            pltpu.SemaphoreType.DMA,
        ],
        compiler_params=pltpu.CompilerParams(use_tc_tiling_on_sc=False),
    )
    def gather_kernel(idx_hbm, table_hbm, out_hbm, ibuf0, ibuf1, rows0, rows1,
                      isem0, isem1, gsem0, gsem1, osem0, osem1):
        nc = lax.axis_size("c")
        wid = lax.axis_index("s") * nc + lax.axis_index("c")
        out_base = wid * b_per_w
        idx_base = wid * idx_rows_per_w

        def idx_copy(pair, ibuf, isem):
            row0 = pl.multiple_of(idx_base + pair * _IPP, 8)
            return pltpu.make_async_copy(
                idx_hbm.at[pl.ds(row0, _IPP), :], ibuf, isem)

        def gathers(ibuf, half, rows, gsem):
            return [
                pltpu.make_async_copy(
                    table_hbm.at[ibuf.at[half * _GPC + j]],
                    rows.at[pl.ds(j * _LANE, _LANE), :],
                    gsem,
                )
                for j in range(_GPC)
            ]

        def writeback(chunk, rows, osem):
            base = pl.multiple_of(out_base + chunk * _CH, _CH)
            return pltpu.make_async_copy(
                rows, out_hbm.at[pl.ds(base, _CH), :], osem)

        idx_copy(0, ibuf0, isem0).start()
        idx_copy(1, ibuf1, isem1).start()
        idx_copy(0, ibuf0, isem0).wait()
        for c in gathers(ibuf0, 0, rows0, gsem0):
            c.start()

        def body(p, carry):
            pb = p % 2

            def run(ibuf, isem, ibuf_n, isem_n):
                for c in gathers(ibuf, 0, rows0, gsem0):
                    c.wait()

                @pl.when(p >= 1)
                def _():
                    writeback(0, rows1, osem1).wait()

                for c in gathers(ibuf, 1, rows1, gsem1):
                    c.start()
                writeback(2 * p, rows0, osem0).start()

                for c in gathers(ibuf, 1, rows1, gsem1):
                    c.wait()

                @pl.when(p + 2 < n_pairs)
                def _():
                    idx_copy(p + 2, ibuf, isem).start()

                writeback(0, rows0, osem0).wait()

                @pl.when(p + 1 < n_pairs)
                def _():
                    idx_copy(p + 1, ibuf_n, isem_n).wait()
                    for c in gathers(ibuf_n, 0, rows0, gsem0):
                        c.start()

                writeback(2 * p + 1, rows1, osem1).start()

            @pl.when(pb == 0)
            def _():
                run(ibuf0, isem0, ibuf1, isem1)

            @pl.when(pb == 1)
            def _():
                run(ibuf1, isem1, ibuf0, isem0)

            return carry

        lax.fori_loop(0, n_pairs, body, 0)
        writeback(0, rows1, osem1).wait()

    return gather_kernel


def kernel(nodes, emb_weight):
    info = plsc.get_sparse_core_info()
    nw = info.num_cores * info.num_subcores
    idx2d = nodes.reshape(_B // _LANE, _LANE)
    out = _make_gather(nw)(idx2d, emb_weight)
    return out.reshape(BATCH, HIST, EMBED_DIM)
